# Initial kernel scaffold; baseline (speedup 1.0000x reference)
import jax
import jax.numpy as jnp
from jax.experimental import pallas as pl


def kernel(x, table):
    raise NotImplementedError("write your pallas kernel here")


# speedup vs baseline: 1.8443x; 1.8443x over previous
import functools

import jax
import jax.numpy as jnp
from jax import lax
from jax.experimental import pallas as pl
from jax.experimental.pallas import tpu as pltpu
from jax.experimental.pallas import tpu_sc as plsc

D_MODEL = 64


@functools.lru_cache(maxsize=None)
def _make_gather(n_rows, d_model):
    info = plsc.get_sparse_core_info()
    nc, ns = info.num_cores, info.num_subcores
    nw = nc * ns
    assert n_rows % nw == 0
    b_per_w = n_rows // nw
    chunk = 1024
    assert b_per_w % chunk == 0
    n_chunks = b_per_w // chunk
    mesh = plsc.VectorSubcoreMesh(core_axis_name="c", subcore_axis_name="s")

    @functools.partial(
        pl.kernel,
        mesh=mesh,
        compiler_params=pltpu.CompilerParams(use_tc_tiling_on_sc=False),
        out_type=jax.ShapeDtypeStruct((n_rows, d_model), jnp.float32),
        scratch_types=[
            pltpu.VMEM((chunk,), jnp.int32),
            pltpu.VMEM((chunk, d_model), jnp.float32),
            pltpu.SemaphoreType.DMA,
        ],
    )
    def gather_kernel(idx_hbm, table_hbm, out_hbm, idx_v, rows_v, sem):
        wid = lax.axis_index("s") * nc + lax.axis_index("c")
        wbase = wid * b_per_w

        def body(i, carry):
            base = wbase + i * chunk
            pltpu.sync_copy(idx_hbm.at[pl.ds(base, chunk)], idx_v)
            pltpu.async_copy(table_hbm.at[idx_v], rows_v, sem).wait()
            pltpu.sync_copy(rows_v, out_hbm.at[pl.ds(base, chunk)])
            return carry

        lax.fori_loop(0, n_chunks, body, 0)

    return gather_kernel


def kernel(x, table):
    b, l = x.shape
    n = b * l
    flat = x.reshape(n).astype(jnp.int32)
    out = _make_gather(n, table.shape[1])(flat, table)
    return out.reshape(b, l, table.shape[1])


# speedup vs baseline: 1.8726x; 1.0154x over previous
import functools

import jax
import jax.numpy as jnp
from jax import lax
from jax.experimental import pallas as pl
from jax.experimental.pallas import tpu as pltpu
from jax.experimental.pallas import tpu_sc as plsc

CHUNK = 800
NBUF = 4


@functools.lru_cache(maxsize=None)
def _make_gather(n_rows, d_model):
    info = plsc.get_sparse_core_info()
    nc, ns = info.num_cores, info.num_subcores
    nw = nc * ns
    assert n_rows % nw == 0
    b_per_w = n_rows // nw
    assert b_per_w % (NBUF * CHUNK) == 0
    n_outer = b_per_w // (NBUF * CHUNK)
    mesh = plsc.VectorSubcoreMesh(core_axis_name="c", subcore_axis_name="s")

    @functools.partial(
        pl.kernel,
        mesh=mesh,
        compiler_params=pltpu.CompilerParams(use_tc_tiling_on_sc=False),
        out_type=jax.ShapeDtypeStruct((n_rows, d_model), jnp.float32),
        scratch_types=[
            pltpu.VMEM((NBUF, CHUNK), jnp.int32),
            pltpu.VMEM((2, CHUNK, d_model), jnp.float32),
            pltpu.SemaphoreType.DMA((NBUF,)),
            pltpu.SemaphoreType.DMA((2,)),
            pltpu.SemaphoreType.DMA((2,)),
        ],
    )
    def gather_kernel(idx_hbm, table_hbm, out_hbm, idx_v, rows_v, sem_idx,
                      sem_gth, sem_st):
        wid = lax.axis_index("s") * nc + lax.axis_index("c")
        wbase = wid * b_per_w

        for b in range(NBUF):
            pltpu.async_copy(
                idx_hbm.at[pl.ds(wbase + b * CHUNK, CHUNK)],
                idx_v.at[b], sem_idx.at[b])

        def outer(j, carry):
            for b in range(NBUF):
                sr = b % 2
                i = j * NBUF + b
                base = wbase + i * CHUNK
                if b >= 2:
                    pltpu.make_async_copy(
                        rows_v.at[sr], out_hbm.at[pl.ds(0, CHUNK)],
                        sem_st.at[sr]).wait()
                else:
                    @pl.when(j > 0)
                    def _():
                        pltpu.make_async_copy(
                            rows_v.at[sr], out_hbm.at[pl.ds(0, CHUNK)],
                            sem_st.at[sr]).wait()
                pltpu.make_async_copy(
                    idx_hbm.at[pl.ds(0, CHUNK)], idx_v.at[b],
                    sem_idx.at[b]).wait()
                gth = pltpu.async_copy(
                    table_hbm.at[idx_v.at[b]], rows_v.at[sr], sem_gth.at[sr])
                gth.wait()
                @pl.when(j < n_outer - 1)
                def _():
                    pltpu.async_copy(
                        idx_hbm.at[pl.ds(base + NBUF * CHUNK, CHUNK)],
                        idx_v.at[b], sem_idx.at[b])
                pltpu.async_copy(
                    rows_v.at[sr], out_hbm.at[pl.ds(base, CHUNK)],
                    sem_st.at[sr])
            return carry

        lax.fori_loop(0, n_outer, outer, 0)
        for sr in range(2):
            pltpu.make_async_copy(
                rows_v.at[sr], out_hbm.at[pl.ds(0, CHUNK)],
                sem_st.at[sr]).wait()

    return gather_kernel


def kernel(x, table):
    b, l = x.shape
    n = b * l
    flat = x.reshape(n).astype(jnp.int32)
    out = _make_gather(n, table.shape[1])(flat, table)
    return out.reshape(b, l, table.shape[1])


# speedup vs baseline: 1.8758x; 1.0017x over previous
import functools

import jax
import jax.numpy as jnp
from jax import lax
from jax.experimental import pallas as pl
from jax.experimental.pallas import tpu as pltpu
from jax.experimental.pallas import tpu_sc as plsc

CHUNK = 400
NSLOT = 4


@functools.lru_cache(maxsize=None)
def _make_gather(n_rows, d_model):
    info = plsc.get_sparse_core_info()
    nc, ns = info.num_cores, info.num_subcores
    nw = nc * ns
    assert n_rows % nw == 0
    b_per_w = n_rows // nw
    assert b_per_w % (NSLOT * CHUNK) == 0
    n_outer = b_per_w // (NSLOT * CHUNK)
    mesh = plsc.VectorSubcoreMesh(core_axis_name="c", subcore_axis_name="s")

    @functools.partial(
        pl.kernel,
        mesh=mesh,
        compiler_params=pltpu.CompilerParams(use_tc_tiling_on_sc=False),
        out_type=jax.ShapeDtypeStruct((n_rows, d_model), jnp.float32),
        scratch_types=[
            pltpu.VMEM((NSLOT, CHUNK), jnp.int32),
            pltpu.VMEM((NSLOT, CHUNK, d_model), jnp.float32),
            pltpu.SemaphoreType.DMA((NSLOT,)),
            pltpu.SemaphoreType.DMA((NSLOT,)),
            pltpu.SemaphoreType.DMA((NSLOT,)),
        ],
    )
    def gather_kernel(idx_hbm, table_hbm, out_hbm, idx_v, rows_v, sem_idx,
                      sem_gth, sem_st):
        wid = lax.axis_index("s") * nc + lax.axis_index("c")
        wbase = wid * b_per_w

        def issue_idx(chunk_id, slot):
            pltpu.async_copy(
                idx_hbm.at[pl.ds(wbase + chunk_id * CHUNK, CHUNK)],
                idx_v.at[slot], sem_idx.at[slot])

        def wait_idx(slot):
            pltpu.make_async_copy(
                idx_hbm.at[pl.ds(0, CHUNK)], idx_v.at[slot],
                sem_idx.at[slot]).wait()

        def issue_gather(slot):
            pltpu.async_copy(
                table_hbm.at[idx_v.at[slot]], rows_v.at[slot],
                sem_gth.at[slot])

        def wait_gather(slot):
            pltpu.make_async_copy(
                table_hbm.at[idx_v.at[slot]], rows_v.at[slot],
                sem_gth.at[slot]).wait()

        def issue_store(chunk_id, slot):
            pltpu.async_copy(
                rows_v.at[slot],
                out_hbm.at[pl.ds(wbase + chunk_id * CHUNK, CHUNK)],
                sem_st.at[slot])

        def wait_store(slot):
            pltpu.make_async_copy(
                rows_v.at[slot], out_hbm.at[pl.ds(0, CHUNK)],
                sem_st.at[slot]).wait()

        for b in range(NSLOT):
            issue_idx(b, b)

        def outer(j, carry):
            for b in range(NSLOT):
                i = j * NSLOT + b
                s = b
                s2 = (b + 2) % NSLOT
                def complete_prev(b=b, i=i, s2=s2):
                    wait_gather(s2)
                    if b >= 2:
                        @pl.when(j < n_outer - 1)
                        def _():
                            issue_idx(i + 2, s2)
                    else:
                        issue_idx(i + 2, s2)
                    issue_store(i - 2, s2)

                if b >= 2:
                    complete_prev()
                else:
                    @pl.when(j > 0)
                    def _(complete_prev=complete_prev):
                        complete_prev()
                @pl.when(j > 0)
                def _():
                    wait_store(s)
                wait_idx(s)
                issue_gather(s)
            return carry

        lax.fori_loop(0, n_outer, outer, 0)

        last = n_outer * NSLOT
        for c in (last - 2, last - 1):
            s = c % NSLOT
            wait_gather(s)
            issue_store(c, s)
        for s in range(NSLOT):
            wait_store(s)

    return gather_kernel


def kernel(x, table):
    b, l = x.shape
    n = b * l
    flat = x.reshape(n).astype(jnp.int32)
    out = _make_gather(n, table.shape[1])(flat, table)
    return out.reshape(b, l, table.shape[1])
